# baseline (speedup 1.0000x reference)
import functools

import jax
import jax.numpy as jnp
from jax import lax
from jax.experimental import pallas as pl
from jax.experimental.pallas import tpu as pltpu
from jax.experimental.pallas import tpu_sc as plsc

_E = 64
_K = 8
_D = 4096
_TA = 1024
_CA = 64

_NW = 32
_TOKENS = 32768
_TW = _TOKENS // _NW
_G = 16
_NG = _TW // _G

_TC = 2048


def _scores_kernel(x_ref, wt_ref, bias_ref, pt_ref):
    for c in range(_TA // _CA):
        sl = slice(c * _CA, (c + 1) * _CA)
        logits = jnp.dot(
            x_ref[sl, :],
            wt_ref[...],
            preferred_element_type=jnp.float32,
            precision=jax.lax.Precision.DEFAULT,
        )
        m = jnp.max(logits, axis=1, keepdims=True)
        e = jnp.exp(logits - m)
        q = e / jnp.sum(e, axis=1, keepdims=True) + bias_ref[...]
        pt_ref[:, sl] = q.T


@jax.jit
def _scores_t(x2, wt, bias):
    return pl.pallas_call(
        _scores_kernel,
        grid=(_TOKENS // _TA,),
        in_specs=[
            pl.BlockSpec((_TA, _D), lambda t: (t, 0)),
            pl.BlockSpec((_D, _E), lambda t: (0, 0)),
            pl.BlockSpec((1, _E), lambda t: (0, 0)),
        ],
        out_specs=pl.BlockSpec((_E, _TA), lambda t: (0, t)),
        out_shape=jax.ShapeDtypeStruct((_E, _TOKENS), jnp.float32),
    )(x2, wt, bias)


def _sc_topk_kernel(qt_hbm, ts_hbm, idx_hbm, q_v, ts_v, idx_v):
    wid = lax.axis_index("s") * 2 + lax.axis_index("c")
    col = wid * _TW

    pltpu.sync_copy(qt_hbm.at[:, pl.ds(col, _TW)], q_v)

    zeros16 = jnp.zeros((16,), jnp.int32)
    neg_inf = jnp.full((16,), -jnp.inf, jnp.float32)

    def group_body(g, carry):
        s = [neg_inf] * _K
        si = [zeros16] * _K
        for e in range(_E):
            carry_v = q_v[e, pl.ds(g * _G, _G)]
            carry_i = jnp.full((16,), e, jnp.int32)
            for j in range(_K):
                swap = carry_v > s[j]
                ns = jnp.where(swap, carry_v, s[j])
                carry_v = jnp.where(swap, s[j], carry_v)
                ni = jnp.where(swap, carry_i, si[j])
                carry_i = jnp.where(swap, si[j], carry_i)
                s[j] = ns
                si[j] = ni
        for k in range(_K):
            ts_v[k, pl.ds(g * _G, _G)] = s[k]
            idx_v[k, pl.ds(g * _G, _G)] = si[k]
        return carry

    lax.fori_loop(0, _NG, group_body, 0)

    pltpu.sync_copy(ts_v, ts_hbm.at[:, pl.ds(col, _TW)])
    pltpu.sync_copy(idx_v, idx_hbm.at[:, pl.ds(col, _TW)])


@jax.jit
def _sc_topk(qt):
    mesh = plsc.VectorSubcoreMesh(core_axis_name="c", subcore_axis_name="s")
    kfn = pl.kernel(
        _sc_topk_kernel,
        mesh=mesh,
        out_type=[
            jax.ShapeDtypeStruct((_K, _TOKENS), jnp.float32),
            jax.ShapeDtypeStruct((_K, _TOKENS), jnp.int32),
        ],
        scratch_types=[
            pltpu.VMEM((_E, _TW), jnp.float32),
            pltpu.VMEM((_K, _TW), jnp.float32),
            pltpu.VMEM((_K, _TW), jnp.int32),
        ],
    )
    return kfn(qt)


def _finish_kernel(tst_ref, idxt_ref, bias_ref, ts_ref, idx_ref, cnt_ref):
    t = pl.program_id(0)
    qs = tst_ref[...]
    ids = idxt_ref[...]

    iota = jax.lax.broadcasted_iota(jnp.int32, (_TC, _E), 1)
    idx_out = ids.T
    idx_ref[...] = idx_out

    acc = jnp.zeros((_TC, _E), jnp.int32)
    bias_row = bias_ref[...]
    bcols = []
    for k in range(_K):
        onehot = iota == idx_out[:, k : k + 1]
        acc += onehot.astype(jnp.int32)
        bcols.append(
            jnp.sum(jnp.where(onehot, bias_row, 0.0), axis=1, keepdims=True)
        )
    ts_ref[...] = qs.T - jnp.concatenate(bcols, axis=1)

    counts = jnp.sum(acc, axis=0, keepdims=True)

    @pl.when(t == 0)
    def _init():
        cnt_ref[...] = jnp.zeros_like(cnt_ref)

    b = t // (8192 // _TC)
    cnt_ref[pl.ds(b, 1), :] += counts


@jax.jit
def _finish(tst, idxt, bias):
    return pl.pallas_call(
        _finish_kernel,
        grid=(_TOKENS // _TC,),
        in_specs=[
            pl.BlockSpec((_K, _TC), lambda t: (0, t)),
            pl.BlockSpec((_K, _TC), lambda t: (0, t)),
            pl.BlockSpec((1, _E), lambda t: (0, 0)),
        ],
        out_specs=[
            pl.BlockSpec((_TC, _K), lambda t: (t, 0)),
            pl.BlockSpec((_TC, _K), lambda t: (t, 0)),
            pl.BlockSpec((4, _E), lambda t: (0, 0)),
        ],
        out_shape=[
            jax.ShapeDtypeStruct((_TOKENS, _K), jnp.float32),
            jax.ShapeDtypeStruct((_TOKENS, _K), jnp.int32),
            jax.ShapeDtypeStruct((4, _E), jnp.int32),
        ],
        compiler_params=pltpu.CompilerParams(
            dimension_semantics=("arbitrary",),
        ),
    )(tst, idxt, bias)


def kernel(x, expert_bias, W):
    B, S, _ = x.shape
    x2 = x.reshape(_TOKENS, _D)
    bias2 = expert_bias.reshape(1, _E)
    qt = _scores_t(x2, W.T, bias2)
    tst, idxt = _sc_topk(qt)
    ts, idx, counts = _finish(tst, idxt, bias2)
    return (
        ts.reshape(B, S, _K),
        idx.reshape(B, S, _K),
        counts,
    )


# speedup vs baseline: 1.6610x; 1.6610x over previous
import functools

import jax
import jax.numpy as jnp
from jax.experimental import pallas as pl
from jax.experimental.pallas import tpu as pltpu

_NUM_EXPERTS = 64
_TOP_K = 8
_DIM = 4096
_T = 1024
_NSPLIT = 1
_C = 64


def _router_kernel(*refs):
    x_refs = refs[:_NSPLIT]
    wt_ref, bias_ref, ts_ref, idx_ref, cnt_ref = refs[_NSPLIT:]
    b = pl.program_id(0)
    t = pl.program_id(1)

    sub = _T // _NSPLIT
    iota = jax.lax.broadcasted_iota(jnp.int32, (_C, _NUM_EXPERTS), 1)
    counts = jnp.zeros((1, _NUM_EXPERTS), dtype=jnp.int32)
    for c in range(_T // _C):
        x_ref = x_refs[c // (sub // _C)]
        r = (c % (sub // _C)) * _C
        sl = slice(c * _C, (c + 1) * _C)
        logits = jnp.dot(
            x_ref[0, r : r + _C, :],
            wt_ref[...],
            preferred_element_type=jnp.float32,
            precision=jax.lax.Precision.DEFAULT,
        )

        m = jnp.max(logits, axis=1, keepdims=True)
        e = jnp.exp(logits - m)
        p = e / jnp.sum(e, axis=1, keepdims=True)

        work = p + bias_ref[...]

        vals = []
        idxs = []
        for _ in range(_TOP_K):
            sel = jnp.argmax(work, axis=1, keepdims=True)
            onehot = iota == sel
            vals.append(jnp.sum(jnp.where(onehot, p, 0.0), axis=1, keepdims=True))
            idxs.append(sel)
            work = jnp.where(onehot, -jnp.inf, work)

        counts += jnp.sum((work == -jnp.inf).astype(jnp.int32), axis=0, keepdims=True)

        ts_ref[0, sl, :] = jnp.concatenate(vals, axis=1)
        idx_ref[0, sl, :] = jnp.concatenate(idxs, axis=1)

    @pl.when(jnp.logical_and(b == 0, t == 0))
    def _init():
        cnt_ref[...] = jnp.zeros_like(cnt_ref)

    cnt_ref[pl.ds(b, 1), :] += counts


@functools.partial(jax.jit, static_argnames=())
def _router(x, expert_bias, wt):
    B, S, D = x.shape
    grid = (B, S // _T)
    sub = _T // _NSPLIT
    x_specs = [
        pl.BlockSpec((1, sub, D), lambda b, t, j=j: (b, t * _NSPLIT + j, 0))
        for j in range(_NSPLIT)
    ]
    return pl.pallas_call(
        _router_kernel,
        grid=grid,
        in_specs=x_specs
        + [
            pl.BlockSpec((D, _NUM_EXPERTS), lambda b, t: (0, 0)),
            pl.BlockSpec((1, _NUM_EXPERTS), lambda b, t: (0, 0)),
        ],
        out_specs=[
            pl.BlockSpec((1, _T, _TOP_K), lambda b, t: (b, t, 0)),
            pl.BlockSpec((1, _T, _TOP_K), lambda b, t: (b, t, 0)),
            pl.BlockSpec((B, _NUM_EXPERTS), lambda b, t: (0, 0)),
        ],
        out_shape=[
            jax.ShapeDtypeStruct((B, S, _TOP_K), jnp.float32),
            jax.ShapeDtypeStruct((B, S, _TOP_K), jnp.int32),
            jax.ShapeDtypeStruct((B, _NUM_EXPERTS), jnp.int32),
        ],
        compiler_params=pltpu.CompilerParams(
            dimension_semantics=("arbitrary", "arbitrary"),
        ),
    )(*([x] * _NSPLIT), wt, expert_bias)


def kernel(x, expert_bias, W):
    top_scores, idx, counts = _router(
        x, expert_bias.reshape(1, _NUM_EXPERTS), W.T
    )
    return (top_scores, idx, counts)


# speedup vs baseline: 1.6942x; 1.0200x over previous
import functools

import jax
import jax.numpy as jnp
from jax.experimental import pallas as pl
from jax.experimental.pallas import tpu as pltpu

_NUM_EXPERTS = 64
_TOP_K = 8
_DIM = 4096
_T = 1024
_NSPLIT = 1
_C = 128


def _router_kernel(*refs):
    x_refs = refs[:_NSPLIT]
    wt_ref, bias_ref, ts_ref, idx_ref, cnt_ref = refs[_NSPLIT:]
    b = pl.program_id(0)
    t = pl.program_id(1)

    sub = _T // _NSPLIT
    iota = jax.lax.broadcasted_iota(jnp.int32, (_C, _NUM_EXPERTS), 1)
    counts = jnp.zeros((1, _NUM_EXPERTS), dtype=jnp.int32)
    for c in range(_T // _C):
        x_ref = x_refs[c // (sub // _C)]
        r = (c % (sub // _C)) * _C
        sl = slice(c * _C, (c + 1) * _C)
        logits = jnp.dot(
            x_ref[0, r : r + _C, :],
            wt_ref[...],
            preferred_element_type=jnp.float32,
            precision=jax.lax.Precision.DEFAULT,
        )

        m = jnp.max(logits, axis=1, keepdims=True)
        e = jnp.exp(logits - m)
        p = e / jnp.sum(e, axis=1, keepdims=True)

        work = p + bias_ref[...]

        vals = []
        idxs = []
        for _ in range(_TOP_K):
            sel = jnp.argmax(work, axis=1, keepdims=True)
            onehot = iota == sel
            vals.append(jnp.sum(jnp.where(onehot, p, 0.0), axis=1, keepdims=True))
            idxs.append(sel)
            work = jnp.where(onehot, -jnp.inf, work)

        counts += jnp.sum((work == -jnp.inf).astype(jnp.int32), axis=0, keepdims=True)

        ts_ref[0, sl, :] = jnp.concatenate(vals, axis=1)
        idx_ref[0, sl, :] = jnp.concatenate(idxs, axis=1)

    @pl.when(jnp.logical_and(b == 0, t == 0))
    def _init():
        cnt_ref[...] = jnp.zeros_like(cnt_ref)

    cnt_ref[pl.ds(b, 1), :] += counts


@functools.partial(jax.jit, static_argnames=())
def _router(x, expert_bias, wt):
    B, S, D = x.shape
    grid = (B, S // _T)
    sub = _T // _NSPLIT
    x_specs = [
        pl.BlockSpec((1, sub, D), lambda b, t, j=j: (b, t * _NSPLIT + j, 0))
        for j in range(_NSPLIT)
    ]
    return pl.pallas_call(
        _router_kernel,
        grid=grid,
        in_specs=x_specs
        + [
            pl.BlockSpec((D, _NUM_EXPERTS), lambda b, t: (0, 0)),
            pl.BlockSpec((1, _NUM_EXPERTS), lambda b, t: (0, 0)),
        ],
        out_specs=[
            pl.BlockSpec((1, _T, _TOP_K), lambda b, t: (b, t, 0)),
            pl.BlockSpec((1, _T, _TOP_K), lambda b, t: (b, t, 0)),
            pl.BlockSpec((B, _NUM_EXPERTS), lambda b, t: (0, 0)),
        ],
        out_shape=[
            jax.ShapeDtypeStruct((B, S, _TOP_K), jnp.float32),
            jax.ShapeDtypeStruct((B, S, _TOP_K), jnp.int32),
            jax.ShapeDtypeStruct((B, _NUM_EXPERTS), jnp.int32),
        ],
        compiler_params=pltpu.CompilerParams(
            dimension_semantics=("arbitrary", "arbitrary"),
        ),
    )(*([x] * _NSPLIT), wt, expert_bias)


def kernel(x, expert_bias, W):
    top_scores, idx, counts = _router(
        x, expert_bias.reshape(1, _NUM_EXPERTS), W.T
    )
    return (top_scores, idx, counts)
